# baseline (speedup 1.0000x reference)
import functools

import jax
import jax.numpy as jnp
from jax import lax
from jax.experimental import pallas as pl
from jax.experimental.pallas import tpu as pltpu
from jax.experimental.pallas import tpu_sc as plsc

IN_PLANES = 128
OUT_PLANES = 128
MID = OUT_PLANES // 2
SHARE = 8
NSAMPLE = 16
B = 2
S = 4096
N = 4096

QB = 256
NBLK = S // QB
NTOT = B * S
CNT = float(NTOT * NSAMPLE)
KV = MID + OUT_PLANES
KVP = 256
EPS = 1e-5

HI = jax.lax.Precision.HIGHEST


def _dot(a, b):
    return lax.dot_general(a, b, (((a.ndim - 1,), (0,)), ((), ())),
                           precision=HI, preferred_element_type=jnp.float32)


def _kv_kernel(p2_ref, xyz2_ref, wkv_ref, bkv_ref, outv_ref, outk_ref):
    mm = _dot(p2_ref[0], wkv_ref[...]) + bkv_ref[...]
    outv_ref[0] = mm[:, 0:OUT_PLANES]
    outk_ref[0] = jnp.concatenate(
        [mm[:, OUT_PLANES:], xyz2_ref[0],
         jnp.zeros((N, 128 - MID - 3), jnp.float32)], axis=1)


def _build_kv_table(points2, xyz2, Wk, bk, Wv, bv):
    wkv = jnp.concatenate([Wv, Wk], axis=1)
    bkv = jnp.concatenate([bv, bk])[None, :]
    tv, tk = pl.pallas_call(
        _kv_kernel,
        grid=(B,),
        in_specs=[
            pl.BlockSpec((1, N, IN_PLANES), lambda b: (b, 0, 0)),
            pl.BlockSpec((1, N, 3), lambda b: (b, 0, 0)),
            pl.BlockSpec((IN_PLANES, KV), lambda b: (0, 0)),
            pl.BlockSpec((1, KV), lambda b: (0, 0)),
        ],
        out_specs=[
            pl.BlockSpec((1, N, 128), lambda b: (b, 0, 0)),
            pl.BlockSpec((1, N, 128), lambda b: (b, 0, 0)),
        ],
        out_shape=[
            jax.ShapeDtypeStruct((B, N, 128), jnp.float32),
            jax.ShapeDtypeStruct((B, N, 128), jnp.float32),
        ],
    )(points2, xyz2, wkv, bkv)
    return tv.reshape(B * N, 128), tk.reshape(B * N, 128)


def _knn_kernel(xyz1_ref, xyz2t_ref, p1_ref, wq_ref, bq_ref,
                idx_ref, xq_ref):
    b = pl.program_id(0)

    q = xyz1_ref[0]
    k_t = xyz2t_ref[0]

    qk = lax.dot_general(q.astype(jnp.bfloat16), k_t.astype(jnp.bfloat16),
                         (((1,), (0,)), ((), ())),
                         preferred_element_type=jnp.float32)
    q2 = (q[:, 0:1] * q[:, 0:1] + q[:, 1:2] * q[:, 1:2]
          + q[:, 2:3] * q[:, 2:3])
    k2 = (k_t[0:1, :] * k_t[0:1, :] + k_t[1:2, :] * k_t[1:2, :]
          + k_t[2:3, :] * k_t[2:3, :])
    dist = -2.0 * qk
    dist = dist + q2
    dist = dist + k2

    iota_n = lax.broadcasted_iota(jnp.int32, (QB, N), 1)
    iota_k = lax.broadcasted_iota(jnp.int32, (QB, NSAMPLE), 1)
    inf = jnp.float32(jnp.inf)

    m0 = jnp.min(dist, axis=1, keepdims=True)

    def topk_body(j, carry):
        dist, m, idx_acc = carry
        amin = jnp.min(jnp.where(dist == m, iota_n, N),
                       axis=1, keepdims=True)
        idx_acc = jnp.where(iota_k == j, amin + b * N, idx_acc)
        dist = jnp.where(iota_n == amin, inf, dist)
        m = jnp.min(dist, axis=1, keepdims=True)
        return dist, m, idx_acc

    _, _, idx_acc = lax.fori_loop(
        0, NSAMPLE, topk_body,
        (dist, m0, jnp.zeros((QB, NSAMPLE), jnp.int32)))

    idx_ref[0] = idx_acc
    xq_ref[0] = _dot(p1_ref[0], wq_ref[...]) + bq_ref[...]


def _run_knn(xyz1, xyz2, points1, Wq, bq):
    xyz2t = jnp.swapaxes(xyz2, 1, 2)
    return pl.pallas_call(
        _knn_kernel,
        grid=(B, NBLK),
        in_specs=[
            pl.BlockSpec((1, QB, 3), lambda b, s: (b, s, 0)),
            pl.BlockSpec((1, 3, N), lambda b, s: (b, 0, 0)),
            pl.BlockSpec((1, QB, IN_PLANES), lambda b, s: (b, s, 0)),
            pl.BlockSpec((IN_PLANES, MID), lambda b, s: (0, 0)),
            pl.BlockSpec((1, MID), lambda b, s: (0, 0)),
        ],
        out_specs=[
            pl.BlockSpec((1, QB, NSAMPLE), lambda b, s: (b, s, 0)),
            pl.BlockSpec((1, QB, MID), lambda b, s: (b, s, 0)),
        ],
        out_shape=[
            jax.ShapeDtypeStruct((B, S, NSAMPLE), jnp.int32),
            jax.ShapeDtypeStruct((B, S, MID), jnp.float32),
        ],
        compiler_params=pltpu.CompilerParams(
            dimension_semantics=("arbitrary", "arbitrary")),
    )(xyz1, xyz2t, points1, Wq, bq[None, :])


def _pr_kernel(kvx_ref, xyz1_ref, pr_ref, st_ref, acc_ref):
    i = pl.program_id(0)

    @pl.when(i == 0)
    def _():
        acc_ref[...] = jnp.zeros_like(acc_ref)

    pr = kvx_ref[:, :, 64:67] - xyz1_ref[...][:, None, :]
    pr_ref[...] = pr

    s1 = jnp.sum(pr, axis=(0, 1)).reshape(1, 3)
    prf = pr.reshape(QB * NSAMPLE, 3)
    s2 = lax.dot_general(prf, prf, (((0,), (0,)), ((), ())),
                         precision=HI, preferred_element_type=jnp.float32)
    z = jnp.zeros((1, 5), jnp.float32)
    acc_ref[...] += jnp.concatenate(
        [jnp.concatenate([s1, z], axis=1),
         jnp.concatenate([s2, jnp.zeros((3, 5), jnp.float32)], axis=1),
         jnp.zeros((4, 8), jnp.float32)], axis=0)

    @pl.when(i == _NQB - 1)
    def _():
        st_ref[...] = acc_ref[...]


def _run_prstats(kv, xyz1f):
    return pl.pallas_call(
        _pr_kernel,
        grid=(_NQB,),
        in_specs=[
            pl.BlockSpec((QB, NSAMPLE, 128), lambda i: (i, 0, 0)),
            pl.BlockSpec((QB, 3), lambda i: (i, 0)),
        ],
        out_specs=[
            pl.BlockSpec((QB, NSAMPLE, 3), lambda i: (i, 0, 0)),
            pl.BlockSpec((8, 8), lambda i: (0, 0)),
        ],
        out_shape=[
            jax.ShapeDtypeStruct((NTOT, NSAMPLE, 3), jnp.float32),
            jax.ShapeDtypeStruct((8, 8), jnp.float32),
        ],
        scratch_shapes=[pltpu.VMEM((8, 8), jnp.float32)],
        compiler_params=pltpu.CompilerParams(
            dimension_semantics=("arbitrary",)),
    )(kv, xyz1f)


GC = 128
NW = 32
ROWS_W = (NTOT * NSAMPLE) // NW
NCH = ROWS_W // GC


def _gather_rows(table, idx_flat):
    mesh = plsc.VectorSubcoreMesh(core_axis_name="c", subcore_axis_name="s")

    @functools.partial(
        pl.kernel,
        mesh=mesh,
        out_type=jax.ShapeDtypeStruct((NTOT * NSAMPLE, 128), jnp.float32),
        scratch_types=[
            pltpu.VMEM((GC,), jnp.int32),
            pltpu.VMEM((GC, 128), jnp.float32),
            pltpu.SemaphoreType.DMA,
        ],
    )
    def gat(table_hbm, idx_hbm, out_hbm, idx_v, rows_v, sem):
        wid = lax.axis_index("s") * 2 + lax.axis_index("c")
        base = wid * ROWS_W

        def body(ch, _):
            off = base + ch * GC
            pltpu.sync_copy(idx_hbm.at[pl.ds(off, GC)], idx_v)
            pltpu.async_copy(table_hbm.at[idx_v], rows_v, sem).wait()
            pltpu.sync_copy(rows_v, out_hbm.at[pl.ds(off, GC)])
            return 0

        lax.fori_loop(0, NCH, body, 0)

    return gat(table, idx_flat)


def _linear_p(pr, st_ref, wp1_ref, bp1_ref, gp_ref, bp_ref, wp2_ref, bp2_ref):
    W = wp1_ref[...]
    s1 = st_ref[0:1, 0:3]
    S2 = st_ref[1:4, 0:3]
    t = _dot(s1, W)
    u = jnp.sum(W * _dot(S2, W), axis=0, keepdims=True)
    b1 = bp1_ref[...]
    mean = t / CNT + b1
    ex2 = u / CNT + 2.0 * b1 * (t / CNT) + b1 * b1
    var = ex2 - mean * mean
    sc = gp_ref[...] * lax.rsqrt(var + EPS)
    sh = bp_ref[...] - mean * sc
    prf = pr.reshape(pr.shape[0] * NSAMPLE, 3)
    pp = _dot(prf, W) + b1
    ppn = jnp.maximum(pp * sc + sh, 0.0)
    p2 = _dot(ppn, wp2_ref[...]) + bp2_ref[...]
    return p2.reshape(pr.shape[0], NSAMPLE, OUT_PLANES)


def _w_pre(kv_ref, xq_ref, pr_ref, st_ref, wp1_ref, bp1_ref, gp_ref, bp_ref,
           wp2_ref, bp2_ref):
    p = _linear_p(pr_ref[...], st_ref, wp1_ref, bp1_ref, gp_ref, bp_ref,
                  wp2_ref, bp2_ref)
    x_k = kv_ref[:, :, 0:MID]
    w = x_k - xq_ref[...][:, None, :] + p[:, :, 0:MID] + p[:, :, MID:]
    return w, p


_SMALL_SPECS = [
    pl.BlockSpec((8, 8), lambda i: (0, 0)),
    pl.BlockSpec((3, 3), lambda i: (0, 0)),
    pl.BlockSpec((1, 3), lambda i: (0, 0)),
    pl.BlockSpec((1, 3), lambda i: (0, 0)),
    pl.BlockSpec((1, 3), lambda i: (0, 0)),
    pl.BlockSpec((3, OUT_PLANES), lambda i: (0, 0)),
    pl.BlockSpec((1, OUT_PLANES), lambda i: (0, 0)),
]
_NQB = NTOT // QB


def _bn2_kernel(kv_ref, xq_ref, pr_ref, st_ref, wp1_ref, bp1_ref, gp_ref,
                bp_ref, wp2_ref, bp2_ref, out_ref, s_ref, ss_ref):
    i = pl.program_id(0)

    @pl.when(i == 0)
    def _():
        s_ref[...] = jnp.zeros_like(s_ref)
        ss_ref[...] = jnp.zeros_like(ss_ref)

    w, _ = _w_pre(kv_ref, xq_ref, pr_ref, st_ref, wp1_ref, bp1_ref, gp_ref,
                  bp_ref, wp2_ref, bp2_ref)
    s_ref[...] += jnp.sum(w, axis=(0, 1))[None, :]
    ss_ref[...] += jnp.sum(w * w, axis=(0, 1))[None, :]

    @pl.when(i == _NQB - 1)
    def _():
        out_ref[...] = jnp.concatenate(
            [s_ref[...], ss_ref[...], jnp.zeros((6, MID), jnp.float32)],
            axis=0)


def _run_bn2(kv, xq, pr, stats, Wp1, bp1, gp, bp, Wp2, bp2):
    return pl.pallas_call(
        _bn2_kernel,
        grid=(_NQB,),
        in_specs=[
            pl.BlockSpec((QB, NSAMPLE, 128), lambda i: (i, 0, 0)),
            pl.BlockSpec((QB, MID), lambda i: (i, 0)),
            pl.BlockSpec((QB, NSAMPLE, 3), lambda i: (i, 0, 0)),
        ] + _SMALL_SPECS,
        out_specs=pl.BlockSpec((8, MID), lambda i: (0, 0)),
        out_shape=jax.ShapeDtypeStruct((8, MID), jnp.float32),
        scratch_shapes=[pltpu.VMEM((1, MID), jnp.float32),
                        pltpu.VMEM((1, MID), jnp.float32)],
        compiler_params=pltpu.CompilerParams(
            dimension_semantics=("arbitrary",)),
    )(kv, xq, pr, stats, Wp1, bp1[None, :], gp[None, :], bp[None, :], Wp2,
      bp2[None, :])


def _bn_vec(stats_ref, g_ref, b_ref, width):
    mean = stats_ref[0:1, :] / CNT
    var = stats_ref[1:2, :] / CNT - mean * mean
    sc = g_ref[...] / jnp.sqrt(var + EPS)
    sh = b_ref[...] - mean * sc
    return sc.reshape(1, 1, width), sh.reshape(1, 1, width)


def _w1_kernel(kv_ref, xq_ref, pr_ref, st_ref, wp1_ref, bp1_ref, gp_ref,
               bp_ref, wp2_ref, bp2_ref, bn2_ref, gw1_ref, bw1g_ref, ww1_ref,
               bw1_ref, w1_ref, out_ref, s_ref, ss_ref):
    i = pl.program_id(0)

    @pl.when(i == 0)
    def _():
        s_ref[...] = jnp.zeros_like(s_ref)
        ss_ref[...] = jnp.zeros_like(ss_ref)

    w, _ = _w_pre(kv_ref, xq_ref, pr_ref, st_ref, wp1_ref, bp1_ref, gp_ref,
                  bp_ref, wp2_ref, bp2_ref)
    sc, sh = _bn_vec(bn2_ref, gw1_ref, bw1g_ref, MID)
    w = jnp.maximum(w * sc + sh, 0.0)
    w1 = _dot(w.reshape(QB * NSAMPLE, MID), ww1_ref[...]) + bw1_ref[...]
    w1 = w1.reshape(QB, NSAMPLE, MID // SHARE)
    w1_ref[...] = w1
    s_ref[...] += jnp.sum(w1, axis=(0, 1))[None, :]
    ss_ref[...] += jnp.sum(w1 * w1, axis=(0, 1))[None, :]

    @pl.when(i == _NQB - 1)
    def _():
        out_ref[...] = jnp.concatenate(
            [s_ref[...], ss_ref[...],
             jnp.zeros((6, MID // SHARE), jnp.float32)], axis=0)


def _run_w1(kv, xq, pr, stats, Wp1, bp1, gp, bp, Wp2, bp2, bn2, gw1, bw1g,
            Ww1, bw1):
    return pl.pallas_call(
        _w1_kernel,
        grid=(_NQB,),
        in_specs=[
            pl.BlockSpec((QB, NSAMPLE, 128), lambda i: (i, 0, 0)),
            pl.BlockSpec((QB, MID), lambda i: (i, 0)),
            pl.BlockSpec((QB, NSAMPLE, 3), lambda i: (i, 0, 0)),
        ] + _SMALL_SPECS + [
            pl.BlockSpec((8, MID), lambda i: (0, 0)),
            pl.BlockSpec((1, MID), lambda i: (0, 0)),
            pl.BlockSpec((1, MID), lambda i: (0, 0)),
            pl.BlockSpec((MID, MID // SHARE), lambda i: (0, 0)),
            pl.BlockSpec((1, MID // SHARE), lambda i: (0, 0)),
        ],
        out_specs=[
            pl.BlockSpec((QB, NSAMPLE, MID // SHARE), lambda i: (i, 0, 0)),
            pl.BlockSpec((8, MID // SHARE), lambda i: (0, 0)),
        ],
        out_shape=[
            jax.ShapeDtypeStruct((NTOT, NSAMPLE, MID // SHARE), jnp.float32),
            jax.ShapeDtypeStruct((8, MID // SHARE), jnp.float32),
        ],
        scratch_shapes=[pltpu.VMEM((1, MID // SHARE), jnp.float32),
                        pltpu.VMEM((1, MID // SHARE), jnp.float32)],
        compiler_params=pltpu.CompilerParams(
            dimension_semantics=("arbitrary",)),
    )(kv, xq, pr, stats, Wp1, bp1[None, :], gp[None, :], bp[None, :], Wp2,
      bp2[None, :], bn2,
      gw1[None, :], bw1g[None, :], Ww1, bw1[None, :])


def _out_kernel(kv_ref, pr_ref, w1_ref, st_ref, wp1_ref, bp1_ref, gp_ref,
                bp_ref, wp2_ref, bp2_ref, bn3_ref, gw2_ref, bw2g_ref, ww2_ref,
                bw2_ref, x_ref):
    p = _linear_p(pr_ref[...], st_ref, wp1_ref, bp1_ref, gp_ref, bp_ref,
                  wp2_ref, bp2_ref)
    sc, sh = _bn_vec(bn3_ref, gw2_ref, bw2g_ref, MID // SHARE)
    w1 = jnp.maximum(w1_ref[...] * sc + sh, 0.0)
    w2 = _dot(w1.reshape(QB * NSAMPLE, MID // SHARE), ww2_ref[...])
    w2 = w2.reshape(QB, NSAMPLE, OUT_PLANES // SHARE) + \
        bw2_ref[...].reshape(1, 1, OUT_PLANES // SHARE)
    m = jnp.max(w2, axis=1, keepdims=True)
    e = jnp.exp(w2 - m)
    w = e * (1.0 / jnp.sum(e, axis=1, keepdims=True))
    wtile = jnp.concatenate([w] * SHARE, axis=2)
    x_v = kv_ref[...]
    x_ref[...] = (x_v + p) * wtile


def _run_out(kv, pr, w1, stats, Wp1, bp1, gp, bp, Wp2, bp2, bn3, gw2, bw2g,
             Ww2, bw2):
    return pl.pallas_call(
        _out_kernel,
        grid=(_NQB,),
        in_specs=[
            pl.BlockSpec((QB, NSAMPLE, 128), lambda i: (i, 0, 0)),
            pl.BlockSpec((QB, NSAMPLE, 3), lambda i: (i, 0, 0)),
            pl.BlockSpec((QB, NSAMPLE, MID // SHARE), lambda i: (i, 0, 0)),
        ] + _SMALL_SPECS + [
            pl.BlockSpec((8, MID // SHARE), lambda i: (0, 0)),
            pl.BlockSpec((1, MID // SHARE), lambda i: (0, 0)),
            pl.BlockSpec((1, MID // SHARE), lambda i: (0, 0)),
            pl.BlockSpec((MID // SHARE, OUT_PLANES // SHARE), lambda i: (0, 0)),
            pl.BlockSpec((1, OUT_PLANES // SHARE), lambda i: (0, 0)),
        ],
        out_specs=pl.BlockSpec((QB, NSAMPLE, OUT_PLANES), lambda i: (i, 0, 0)),
        out_shape=jax.ShapeDtypeStruct((NTOT, NSAMPLE, OUT_PLANES),
                                       jnp.float32),
    )(kv, pr, w1, stats, Wp1, bp1[None, :], gp[None, :], bp[None, :], Wp2,
      bp2[None, :], bn3,
      gw2[None, :], bw2g[None, :], Ww2, bw2[None, :])


def kernel(xyz1, xyz2, points1, points2, Wq, bq, Wk, bk, Wv, bv, Wp1, bp1,
           gp, bp, Wp2, bp2, gw1, bw1g, Ww1, bw1, gw2, bw2g, Ww2, bw2):
    t_v, t_kx = _build_kv_table(points2, xyz2, Wk, bk, Wv, bv)
    idx, xq = _run_knn(xyz1, xyz2, points1, Wq, bq)
    idx_flat = idx.reshape(NTOT * NSAMPLE)

    g_kx = _gather_rows(t_kx, idx_flat).reshape(NTOT, NSAMPLE, 128)
    g_v = _gather_rows(t_v, idx_flat).reshape(NTOT, NSAMPLE, 128)

    pr_f, stats = _run_prstats(g_kx, xyz1.reshape(NTOT, 3))
    xq_f = xq.reshape(NTOT, MID)

    bn2 = _run_bn2(g_kx, xq_f, pr_f, stats, Wp1, bp1, gp, bp, Wp2, bp2)
    w1, bn3 = _run_w1(g_kx, xq_f, pr_f, stats, Wp1, bp1, gp, bp, Wp2, bp2,
                      bn2, gw1, bw1g, Ww1, bw1)
    x = _run_out(g_v, pr_f, w1, stats, Wp1, bp1, gp, bp, Wp2, bp2, bn3,
                 gw2, bw2g, Ww2, bw2)
    return (x, pr_f)


# speedup vs baseline: 1.0340x; 1.0340x over previous
import functools

import jax
import jax.numpy as jnp
from jax import lax
from jax.experimental import pallas as pl
from jax.experimental.pallas import tpu as pltpu
from jax.experimental.pallas import tpu_sc as plsc

IN_PLANES = 128
OUT_PLANES = 128
MID = OUT_PLANES // 2
SHARE = 8
NSAMPLE = 16
B = 2
S = 4096
N = 4096

QB = 256
NBLK = S // QB
NTOT = B * S
CNT = float(NTOT * NSAMPLE)
KV = MID + OUT_PLANES
KVP = 256
EPS = 1e-5

HI = jax.lax.Precision.HIGHEST


def _dot(a, b):
    return lax.dot_general(a, b, (((a.ndim - 1,), (0,)), ((), ())),
                           precision=HI, preferred_element_type=jnp.float32)


def _kv_kernel(p2_ref, xyz2_ref, wkv_ref, bkv_ref, outv_ref, outk_ref):
    mm = _dot(p2_ref[0], wkv_ref[...]) + bkv_ref[...]
    outv_ref[0] = mm[:, 0:OUT_PLANES]
    outk_ref[0] = jnp.concatenate(
        [mm[:, OUT_PLANES:], xyz2_ref[0],
         jnp.zeros((N, 128 - MID - 3), jnp.float32)], axis=1)


def _build_kv_table(points2, xyz2, Wk, bk, Wv, bv):
    wkv = jnp.concatenate([Wv, Wk], axis=1)
    bkv = jnp.concatenate([bv, bk])[None, :]
    tv, tk = pl.pallas_call(
        _kv_kernel,
        grid=(B,),
        in_specs=[
            pl.BlockSpec((1, N, IN_PLANES), lambda b: (b, 0, 0)),
            pl.BlockSpec((1, N, 3), lambda b: (b, 0, 0)),
            pl.BlockSpec((IN_PLANES, KV), lambda b: (0, 0)),
            pl.BlockSpec((1, KV), lambda b: (0, 0)),
        ],
        out_specs=[
            pl.BlockSpec((1, N, 128), lambda b: (b, 0, 0)),
            pl.BlockSpec((1, N, 128), lambda b: (b, 0, 0)),
        ],
        out_shape=[
            jax.ShapeDtypeStruct((B, N, 128), jnp.float32),
            jax.ShapeDtypeStruct((B, N, 128), jnp.float32),
        ],
    )(points2, xyz2, wkv, bkv)
    return tv.reshape(B * N, 128), tk.reshape(B * N, 128)


def _knn_kernel(xyz1_ref, xyz2t_ref, p1_ref, wq_ref, bq_ref,
                idx_ref, xq_ref):
    b = pl.program_id(0)

    q = xyz1_ref[0]
    k_t = xyz2t_ref[0]

    qk = lax.dot_general(q.astype(jnp.bfloat16), k_t.astype(jnp.bfloat16),
                         (((1,), (0,)), ((), ())),
                         preferred_element_type=jnp.float32)
    q2 = (q[:, 0:1] * q[:, 0:1] + q[:, 1:2] * q[:, 1:2]
          + q[:, 2:3] * q[:, 2:3])
    k2 = (k_t[0:1, :] * k_t[0:1, :] + k_t[1:2, :] * k_t[1:2, :]
          + k_t[2:3, :] * k_t[2:3, :])
    dist = -2.0 * qk
    dist = dist + q2
    dist = dist + k2

    iota_n = lax.broadcasted_iota(jnp.int32, (QB, N), 1)
    iota_k = lax.broadcasted_iota(jnp.int32, (QB, NSAMPLE), 1)
    inf = jnp.float32(jnp.inf)

    def topk_body(j, carry):
        dist, idx_acc = carry
        m = jnp.min(dist, axis=1, keepdims=True)
        amin = jnp.min(jnp.where(dist == m, iota_n, N),
                       axis=1, keepdims=True)
        idx_acc = jnp.where(iota_k == j, amin + b * N, idx_acc)
        dist = jnp.where(iota_n == amin, inf, dist)
        return dist, idx_acc

    _, idx_acc = lax.fori_loop(
        0, NSAMPLE, topk_body,
        (dist, jnp.zeros((QB, NSAMPLE), jnp.int32)))

    idx_ref[0] = idx_acc
    xq_ref[0] = _dot(p1_ref[0], wq_ref[...]) + bq_ref[...]


def _run_knn(xyz1, xyz2, points1, Wq, bq):
    xyz2t = jnp.swapaxes(xyz2, 1, 2)
    return pl.pallas_call(
        _knn_kernel,
        grid=(B, NBLK),
        in_specs=[
            pl.BlockSpec((1, QB, 3), lambda b, s: (b, s, 0)),
            pl.BlockSpec((1, 3, N), lambda b, s: (b, 0, 0)),
            pl.BlockSpec((1, QB, IN_PLANES), lambda b, s: (b, s, 0)),
            pl.BlockSpec((IN_PLANES, MID), lambda b, s: (0, 0)),
            pl.BlockSpec((1, MID), lambda b, s: (0, 0)),
        ],
        out_specs=[
            pl.BlockSpec((1, QB, NSAMPLE), lambda b, s: (b, s, 0)),
            pl.BlockSpec((1, QB, MID), lambda b, s: (b, s, 0)),
        ],
        out_shape=[
            jax.ShapeDtypeStruct((B, S, NSAMPLE), jnp.int32),
            jax.ShapeDtypeStruct((B, S, MID), jnp.float32),
        ],
        compiler_params=pltpu.CompilerParams(
            dimension_semantics=("arbitrary", "arbitrary")),
    )(xyz1, xyz2t, points1, Wq, bq[None, :])


def _pr_kernel(kvx_ref, xyz1_ref, pr_ref, st_ref, acc_ref):
    i = pl.program_id(0)

    @pl.when(i == 0)
    def _():
        acc_ref[...] = jnp.zeros_like(acc_ref)

    pr = kvx_ref[:, :, 64:67] - xyz1_ref[...][:, None, :]
    pr_ref[...] = pr

    s1 = jnp.sum(pr, axis=(0, 1)).reshape(1, 3)
    prf = pr.reshape(QB * NSAMPLE, 3)
    s2 = lax.dot_general(prf, prf, (((0,), (0,)), ((), ())),
                         precision=HI, preferred_element_type=jnp.float32)
    z = jnp.zeros((1, 5), jnp.float32)
    acc_ref[...] += jnp.concatenate(
        [jnp.concatenate([s1, z], axis=1),
         jnp.concatenate([s2, jnp.zeros((3, 5), jnp.float32)], axis=1),
         jnp.zeros((4, 8), jnp.float32)], axis=0)

    @pl.when(i == _NQB - 1)
    def _():
        st_ref[...] = acc_ref[...]


def _run_prstats(kv, xyz1f):
    return pl.pallas_call(
        _pr_kernel,
        grid=(_NQB,),
        in_specs=[
            pl.BlockSpec((QB, NSAMPLE, 128), lambda i: (i, 0, 0)),
            pl.BlockSpec((QB, 3), lambda i: (i, 0)),
        ],
        out_specs=[
            pl.BlockSpec((QB, NSAMPLE, 3), lambda i: (i, 0, 0)),
            pl.BlockSpec((8, 8), lambda i: (0, 0)),
        ],
        out_shape=[
            jax.ShapeDtypeStruct((NTOT, NSAMPLE, 3), jnp.float32),
            jax.ShapeDtypeStruct((8, 8), jnp.float32),
        ],
        scratch_shapes=[pltpu.VMEM((8, 8), jnp.float32)],
        compiler_params=pltpu.CompilerParams(
            dimension_semantics=("arbitrary",)),
    )(kv, xyz1f)


GC = 128
NW = 32
ROWS_W = (NTOT * NSAMPLE) // NW
NCH = ROWS_W // GC


def _gather_rows(table, idx_flat):
    mesh = plsc.VectorSubcoreMesh(core_axis_name="c", subcore_axis_name="s")

    @functools.partial(
        pl.kernel,
        mesh=mesh,
        out_type=jax.ShapeDtypeStruct((NTOT * NSAMPLE, 128), jnp.float32),
        scratch_types=[
            pltpu.VMEM((GC,), jnp.int32),
            pltpu.VMEM((GC, 128), jnp.float32),
            pltpu.SemaphoreType.DMA,
        ],
    )
    def gat(table_hbm, idx_hbm, out_hbm, idx_v, rows_v, sem):
        wid = lax.axis_index("s") * 2 + lax.axis_index("c")
        base = wid * ROWS_W

        def body(ch, _):
            off = base + ch * GC
            pltpu.sync_copy(idx_hbm.at[pl.ds(off, GC)], idx_v)
            pltpu.async_copy(table_hbm.at[idx_v], rows_v, sem).wait()
            pltpu.sync_copy(rows_v, out_hbm.at[pl.ds(off, GC)])
            return 0

        lax.fori_loop(0, NCH, body, 0)

    return gat(table, idx_flat)


def _linear_p(pr, st_ref, wp1_ref, bp1_ref, gp_ref, bp_ref, wp2_ref, bp2_ref):
    W = wp1_ref[...]
    s1 = st_ref[0:1, 0:3]
    S2 = st_ref[1:4, 0:3]
    t = _dot(s1, W)
    u = jnp.sum(W * _dot(S2, W), axis=0, keepdims=True)
    b1 = bp1_ref[...]
    mean = t / CNT + b1
    ex2 = u / CNT + 2.0 * b1 * (t / CNT) + b1 * b1
    var = ex2 - mean * mean
    sc = gp_ref[...] * lax.rsqrt(var + EPS)
    sh = bp_ref[...] - mean * sc
    prf = pr.reshape(pr.shape[0] * NSAMPLE, 3)
    pp = _dot(prf, W) + b1
    ppn = jnp.maximum(pp * sc + sh, 0.0)
    p2 = _dot(ppn, wp2_ref[...]) + bp2_ref[...]
    return p2.reshape(pr.shape[0], NSAMPLE, OUT_PLANES)


def _w_pre(kv_ref, xq_ref, pr_ref, st_ref, wp1_ref, bp1_ref, gp_ref, bp_ref,
           wp2_ref, bp2_ref):
    p = _linear_p(pr_ref[...], st_ref, wp1_ref, bp1_ref, gp_ref, bp_ref,
                  wp2_ref, bp2_ref)
    x_k = kv_ref[:, :, 0:MID]
    w = x_k - xq_ref[...][:, None, :] + p[:, :, 0:MID] + p[:, :, MID:]
    return w, p


_SMALL_SPECS = [
    pl.BlockSpec((8, 8), lambda i: (0, 0)),
    pl.BlockSpec((3, 3), lambda i: (0, 0)),
    pl.BlockSpec((1, 3), lambda i: (0, 0)),
    pl.BlockSpec((1, 3), lambda i: (0, 0)),
    pl.BlockSpec((1, 3), lambda i: (0, 0)),
    pl.BlockSpec((3, OUT_PLANES), lambda i: (0, 0)),
    pl.BlockSpec((1, OUT_PLANES), lambda i: (0, 0)),
]
_NQB = NTOT // QB


def _bn2_kernel(kv_ref, xq_ref, pr_ref, st_ref, wp1_ref, bp1_ref, gp_ref,
                bp_ref, wp2_ref, bp2_ref, out_ref, s_ref, ss_ref):
    i = pl.program_id(0)

    @pl.when(i == 0)
    def _():
        s_ref[...] = jnp.zeros_like(s_ref)
        ss_ref[...] = jnp.zeros_like(ss_ref)

    w, _ = _w_pre(kv_ref, xq_ref, pr_ref, st_ref, wp1_ref, bp1_ref, gp_ref,
                  bp_ref, wp2_ref, bp2_ref)
    s_ref[...] += jnp.sum(w, axis=(0, 1))[None, :]
    ss_ref[...] += jnp.sum(w * w, axis=(0, 1))[None, :]

    @pl.when(i == _NQB - 1)
    def _():
        out_ref[...] = jnp.concatenate(
            [s_ref[...], ss_ref[...], jnp.zeros((6, MID), jnp.float32)],
            axis=0)


def _run_bn2(kv, xq, pr, stats, Wp1, bp1, gp, bp, Wp2, bp2):
    return pl.pallas_call(
        _bn2_kernel,
        grid=(_NQB,),
        in_specs=[
            pl.BlockSpec((QB, NSAMPLE, 128), lambda i: (i, 0, 0)),
            pl.BlockSpec((QB, MID), lambda i: (i, 0)),
            pl.BlockSpec((QB, NSAMPLE, 3), lambda i: (i, 0, 0)),
        ] + _SMALL_SPECS,
        out_specs=pl.BlockSpec((8, MID), lambda i: (0, 0)),
        out_shape=jax.ShapeDtypeStruct((8, MID), jnp.float32),
        scratch_shapes=[pltpu.VMEM((1, MID), jnp.float32),
                        pltpu.VMEM((1, MID), jnp.float32)],
        compiler_params=pltpu.CompilerParams(
            dimension_semantics=("arbitrary",)),
    )(kv, xq, pr, stats, Wp1, bp1[None, :], gp[None, :], bp[None, :], Wp2,
      bp2[None, :])


def _bn_vec(stats_ref, g_ref, b_ref, width):
    mean = stats_ref[0:1, :] / CNT
    var = stats_ref[1:2, :] / CNT - mean * mean
    sc = g_ref[...] / jnp.sqrt(var + EPS)
    sh = b_ref[...] - mean * sc
    return sc.reshape(1, 1, width), sh.reshape(1, 1, width)


def _w1_kernel(kv_ref, xq_ref, pr_ref, st_ref, wp1_ref, bp1_ref, gp_ref,
               bp_ref, wp2_ref, bp2_ref, bn2_ref, gw1_ref, bw1g_ref, ww1_ref,
               bw1_ref, w1_ref, out_ref, s_ref, ss_ref):
    i = pl.program_id(0)

    @pl.when(i == 0)
    def _():
        s_ref[...] = jnp.zeros_like(s_ref)
        ss_ref[...] = jnp.zeros_like(ss_ref)

    w, _ = _w_pre(kv_ref, xq_ref, pr_ref, st_ref, wp1_ref, bp1_ref, gp_ref,
                  bp_ref, wp2_ref, bp2_ref)
    sc, sh = _bn_vec(bn2_ref, gw1_ref, bw1g_ref, MID)
    w = jnp.maximum(w * sc + sh, 0.0)
    w1 = _dot(w.reshape(QB * NSAMPLE, MID), ww1_ref[...]) + bw1_ref[...]
    w1 = w1.reshape(QB, NSAMPLE, MID // SHARE)
    w1_ref[...] = w1
    s_ref[...] += jnp.sum(w1, axis=(0, 1))[None, :]
    ss_ref[...] += jnp.sum(w1 * w1, axis=(0, 1))[None, :]

    @pl.when(i == _NQB - 1)
    def _():
        out_ref[...] = jnp.concatenate(
            [s_ref[...], ss_ref[...],
             jnp.zeros((6, MID // SHARE), jnp.float32)], axis=0)


def _run_w1(kv, xq, pr, stats, Wp1, bp1, gp, bp, Wp2, bp2, bn2, gw1, bw1g,
            Ww1, bw1):
    return pl.pallas_call(
        _w1_kernel,
        grid=(_NQB,),
        in_specs=[
            pl.BlockSpec((QB, NSAMPLE, 128), lambda i: (i, 0, 0)),
            pl.BlockSpec((QB, MID), lambda i: (i, 0)),
            pl.BlockSpec((QB, NSAMPLE, 3), lambda i: (i, 0, 0)),
        ] + _SMALL_SPECS + [
            pl.BlockSpec((8, MID), lambda i: (0, 0)),
            pl.BlockSpec((1, MID), lambda i: (0, 0)),
            pl.BlockSpec((1, MID), lambda i: (0, 0)),
            pl.BlockSpec((MID, MID // SHARE), lambda i: (0, 0)),
            pl.BlockSpec((1, MID // SHARE), lambda i: (0, 0)),
        ],
        out_specs=[
            pl.BlockSpec((QB, NSAMPLE, MID // SHARE), lambda i: (i, 0, 0)),
            pl.BlockSpec((8, MID // SHARE), lambda i: (0, 0)),
        ],
        out_shape=[
            jax.ShapeDtypeStruct((NTOT, NSAMPLE, MID // SHARE), jnp.float32),
            jax.ShapeDtypeStruct((8, MID // SHARE), jnp.float32),
        ],
        scratch_shapes=[pltpu.VMEM((1, MID // SHARE), jnp.float32),
                        pltpu.VMEM((1, MID // SHARE), jnp.float32)],
        compiler_params=pltpu.CompilerParams(
            dimension_semantics=("arbitrary",)),
    )(kv, xq, pr, stats, Wp1, bp1[None, :], gp[None, :], bp[None, :], Wp2,
      bp2[None, :], bn2,
      gw1[None, :], bw1g[None, :], Ww1, bw1[None, :])


def _out_kernel(kv_ref, pr_ref, w1_ref, st_ref, wp1_ref, bp1_ref, gp_ref,
                bp_ref, wp2_ref, bp2_ref, bn3_ref, gw2_ref, bw2g_ref, ww2_ref,
                bw2_ref, x_ref):
    p = _linear_p(pr_ref[...], st_ref, wp1_ref, bp1_ref, gp_ref, bp_ref,
                  wp2_ref, bp2_ref)
    sc, sh = _bn_vec(bn3_ref, gw2_ref, bw2g_ref, MID // SHARE)
    w1 = jnp.maximum(w1_ref[...] * sc + sh, 0.0)
    w2 = _dot(w1.reshape(QB * NSAMPLE, MID // SHARE), ww2_ref[...])
    w2 = w2.reshape(QB, NSAMPLE, OUT_PLANES // SHARE) + \
        bw2_ref[...].reshape(1, 1, OUT_PLANES // SHARE)
    m = jnp.max(w2, axis=1, keepdims=True)
    e = jnp.exp(w2 - m)
    w = e * (1.0 / jnp.sum(e, axis=1, keepdims=True))
    wtile = jnp.concatenate([w] * SHARE, axis=2)
    x_v = kv_ref[...]
    x_ref[...] = (x_v + p) * wtile


def _run_out(kv, pr, w1, stats, Wp1, bp1, gp, bp, Wp2, bp2, bn3, gw2, bw2g,
             Ww2, bw2):
    return pl.pallas_call(
        _out_kernel,
        grid=(_NQB,),
        in_specs=[
            pl.BlockSpec((QB, NSAMPLE, 128), lambda i: (i, 0, 0)),
            pl.BlockSpec((QB, NSAMPLE, 3), lambda i: (i, 0, 0)),
            pl.BlockSpec((QB, NSAMPLE, MID // SHARE), lambda i: (i, 0, 0)),
        ] + _SMALL_SPECS + [
            pl.BlockSpec((8, MID // SHARE), lambda i: (0, 0)),
            pl.BlockSpec((1, MID // SHARE), lambda i: (0, 0)),
            pl.BlockSpec((1, MID // SHARE), lambda i: (0, 0)),
            pl.BlockSpec((MID // SHARE, OUT_PLANES // SHARE), lambda i: (0, 0)),
            pl.BlockSpec((1, OUT_PLANES // SHARE), lambda i: (0, 0)),
        ],
        out_specs=pl.BlockSpec((QB, NSAMPLE, OUT_PLANES), lambda i: (i, 0, 0)),
        out_shape=jax.ShapeDtypeStruct((NTOT, NSAMPLE, OUT_PLANES),
                                       jnp.float32),
    )(kv, pr, w1, stats, Wp1, bp1[None, :], gp[None, :], bp[None, :], Wp2,
      bp2[None, :], bn3,
      gw2[None, :], bw2g[None, :], Ww2, bw2[None, :])


def kernel(xyz1, xyz2, points1, points2, Wq, bq, Wk, bk, Wv, bv, Wp1, bp1,
           gp, bp, Wp2, bp2, gw1, bw1g, Ww1, bw1, gw2, bw2g, Ww2, bw2):
    t_v, t_kx = _build_kv_table(points2, xyz2, Wk, bk, Wv, bv)
    idx, xq = _run_knn(xyz1, xyz2, points1, Wq, bq)
    idx_flat = idx.reshape(NTOT * NSAMPLE)

    g_kx = _gather_rows(t_kx, idx_flat).reshape(NTOT, NSAMPLE, 128)
    g_v = _gather_rows(t_v, idx_flat).reshape(NTOT, NSAMPLE, 128)

    pr_f, stats = _run_prstats(g_kx, xyz1.reshape(NTOT, 3))
    xq_f = xq.reshape(NTOT, MID)

    bn2 = _run_bn2(g_kx, xq_f, pr_f, stats, Wp1, bp1, gp, bp, Wp2, bp2)
    w1, bn3 = _run_w1(g_kx, xq_f, pr_f, stats, Wp1, bp1, gp, bp, Wp2, bp2,
                      bn2, gw1, bw1g, Ww1, bw1)
    x = _run_out(g_v, pr_f, w1, stats, Wp1, bp1, gp, bp, Wp2, bp2, bn3,
                 gw2, bw2g, Ww2, bw2)
    return (x, pr_f)


# speedup vs baseline: 1.1451x; 1.1074x over previous
import functools

import jax
import jax.numpy as jnp
from jax import lax
from jax.experimental import pallas as pl
from jax.experimental.pallas import tpu as pltpu
from jax.experimental.pallas import tpu_sc as plsc

IN_PLANES = 128
OUT_PLANES = 128
MID = OUT_PLANES // 2
SHARE = 8
NSAMPLE = 16
B = 2
S = 4096
N = 4096

QB = 256
NBLK = S // QB
NTOT = B * S
CNT = float(NTOT * NSAMPLE)
KV = MID + OUT_PLANES
KVP = 256
EPS = 1e-5

HI = jax.lax.Precision.HIGHEST


def _dot(a, b):
    return lax.dot_general(a, b, (((a.ndim - 1,), (0,)), ((), ())),
                           precision=HI, preferred_element_type=jnp.float32)


def _kv_kernel(p2_ref, xyz2_ref, wkv_ref, bkv_ref, outv_ref, outk_ref):
    mm = _dot(p2_ref[0], wkv_ref[...]) + bkv_ref[...]
    outv_ref[0] = mm[:, 0:OUT_PLANES]
    outk_ref[0] = jnp.concatenate(
        [mm[:, OUT_PLANES:], xyz2_ref[0],
         jnp.zeros((N, 128 - MID - 3), jnp.float32)], axis=1)


def _build_kv_table(points2, xyz2, Wk, bk, Wv, bv):
    wkv = jnp.concatenate([Wv, Wk], axis=1)
    bkv = jnp.concatenate([bv, bk])[None, :]
    tv, tk = pl.pallas_call(
        _kv_kernel,
        grid=(B,),
        in_specs=[
            pl.BlockSpec((1, N, IN_PLANES), lambda b: (b, 0, 0)),
            pl.BlockSpec((1, N, 3), lambda b: (b, 0, 0)),
            pl.BlockSpec((IN_PLANES, KV), lambda b: (0, 0)),
            pl.BlockSpec((1, KV), lambda b: (0, 0)),
        ],
        out_specs=[
            pl.BlockSpec((1, N, 128), lambda b: (b, 0, 0)),
            pl.BlockSpec((1, N, 128), lambda b: (b, 0, 0)),
        ],
        out_shape=[
            jax.ShapeDtypeStruct((B, N, 128), jnp.float32),
            jax.ShapeDtypeStruct((B, N, 128), jnp.float32),
        ],
    )(points2, xyz2, wkv, bkv)
    return tv.reshape(B * N, 128), tk.reshape(B * N, 128)


def _knn_kernel(xyz1_ref, xyz2t_ref, p1_ref, wq_ref, bq_ref,
                idx_ref, xq_ref):
    b = pl.program_id(0)

    q = xyz1_ref[0]
    k_t = xyz2t_ref[0]

    qk = lax.dot_general(q.astype(jnp.bfloat16), k_t.astype(jnp.bfloat16),
                         (((1,), (0,)), ((), ())),
                         preferred_element_type=jnp.float32)
    q2 = (q[:, 0:1] * q[:, 0:1] + q[:, 1:2] * q[:, 1:2]
          + q[:, 2:3] * q[:, 2:3])
    k2 = (k_t[0:1, :] * k_t[0:1, :] + k_t[1:2, :] * k_t[1:2, :]
          + k_t[2:3, :] * k_t[2:3, :])
    dist = -2.0 * qk
    dist = dist + q2
    dist = dist + k2

    iota_n = lax.broadcasted_iota(jnp.int32, (QB, N), 1)
    iota_k = lax.broadcasted_iota(jnp.int32, (QB, NSAMPLE), 1)
    inf = jnp.float32(jnp.inf)

    def topk_body(j, carry):
        dist, idx_acc = carry
        m = jnp.min(dist, axis=1, keepdims=True)
        amin = jnp.min(jnp.where(dist == m, iota_n, N),
                       axis=1, keepdims=True)
        idx_acc = jnp.where(iota_k == j, amin + b * N, idx_acc)
        dist = jnp.where(iota_n == amin, inf, dist)
        return dist, idx_acc

    _, idx_acc = lax.fori_loop(
        0, NSAMPLE, topk_body,
        (dist, jnp.zeros((QB, NSAMPLE), jnp.int32)))

    idx_ref[0] = idx_acc
    xq_ref[0] = _dot(p1_ref[0], wq_ref[...]) + bq_ref[...]


def _run_knn(xyz1, xyz2, points1, Wq, bq):
    xyz2t = jnp.swapaxes(xyz2, 1, 2)
    return pl.pallas_call(
        _knn_kernel,
        grid=(B, NBLK),
        in_specs=[
            pl.BlockSpec((1, QB, 3), lambda b, s: (b, s, 0)),
            pl.BlockSpec((1, 3, N), lambda b, s: (b, 0, 0)),
            pl.BlockSpec((1, QB, IN_PLANES), lambda b, s: (b, s, 0)),
            pl.BlockSpec((IN_PLANES, MID), lambda b, s: (0, 0)),
            pl.BlockSpec((1, MID), lambda b, s: (0, 0)),
        ],
        out_specs=[
            pl.BlockSpec((1, QB, NSAMPLE), lambda b, s: (b, s, 0)),
            pl.BlockSpec((1, QB, MID), lambda b, s: (b, s, 0)),
        ],
        out_shape=[
            jax.ShapeDtypeStruct((B, S, NSAMPLE), jnp.int32),
            jax.ShapeDtypeStruct((B, S, MID), jnp.float32),
        ],
        compiler_params=pltpu.CompilerParams(
            dimension_semantics=("arbitrary", "arbitrary")),
    )(xyz1, xyz2t, points1, Wq, bq[None, :])


def _pr_kernel(kvx_ref, xyz1_ref, pr_ref, st_ref, acc_ref):
    i = pl.program_id(0)

    @pl.when(i == 0)
    def _():
        acc_ref[...] = jnp.zeros_like(acc_ref)

    pr = kvx_ref[:, :, 64:67] - xyz1_ref[...][:, None, :]
    pr_ref[...] = pr

    s1 = jnp.sum(pr, axis=(0, 1)).reshape(1, 3)
    prf = pr.reshape(QB * NSAMPLE, 3)
    s2 = lax.dot_general(prf, prf, (((0,), (0,)), ((), ())),
                         precision=HI, preferred_element_type=jnp.float32)
    z = jnp.zeros((1, 5), jnp.float32)
    acc_ref[...] += jnp.concatenate(
        [jnp.concatenate([s1, z], axis=1),
         jnp.concatenate([s2, jnp.zeros((3, 5), jnp.float32)], axis=1),
         jnp.zeros((4, 8), jnp.float32)], axis=0)

    @pl.when(i == _NQB - 1)
    def _():
        st_ref[...] = acc_ref[...]


def _run_prstats(kv, xyz1f):
    return pl.pallas_call(
        _pr_kernel,
        grid=(_NQB,),
        in_specs=[
            pl.BlockSpec((QB, NSAMPLE, 128), lambda i: (i, 0, 0)),
            pl.BlockSpec((QB, 3), lambda i: (i, 0)),
        ],
        out_specs=[
            pl.BlockSpec((QB, NSAMPLE, 3), lambda i: (i, 0, 0)),
            pl.BlockSpec((8, 8), lambda i: (0, 0)),
        ],
        out_shape=[
            jax.ShapeDtypeStruct((NTOT, NSAMPLE, 3), jnp.float32),
            jax.ShapeDtypeStruct((8, 8), jnp.float32),
        ],
        scratch_shapes=[pltpu.VMEM((8, 8), jnp.float32)],
        compiler_params=pltpu.CompilerParams(
            dimension_semantics=("arbitrary",)),
    )(kv, xyz1f)


GC = 128
NW = 32
ROWS_W = (NTOT * NSAMPLE) // NW
NCH = ROWS_W // GC


def _gather_rows(table, idx_flat):
    mesh = plsc.VectorSubcoreMesh(core_axis_name="c", subcore_axis_name="s")

    @functools.partial(
        pl.kernel,
        mesh=mesh,
        out_type=jax.ShapeDtypeStruct((NTOT * NSAMPLE, 128), jnp.float32),
        scratch_types=[
            pltpu.VMEM((GC,), jnp.int32),
            pltpu.VMEM((GC, 128), jnp.float32),
            pltpu.SemaphoreType.DMA,
        ],
    )
    def gat(table_hbm, idx_hbm, out_hbm, idx_v, rows_v, sem):
        wid = lax.axis_index("s") * 2 + lax.axis_index("c")
        base = wid * ROWS_W

        def body(ch, _):
            off = base + ch * GC
            pltpu.sync_copy(idx_hbm.at[pl.ds(off, GC)], idx_v)
            pltpu.async_copy(table_hbm.at[idx_v], rows_v, sem).wait()
            pltpu.sync_copy(rows_v, out_hbm.at[pl.ds(off, GC)])
            return 0

        lax.fori_loop(0, NCH, body, 0)

    return gat(table, idx_flat)


def _linear_p(pr, st_ref, wp1_ref, bp1_ref, gp_ref, bp_ref, wp2_ref, bp2_ref):
    W = wp1_ref[...]
    s1 = st_ref[0:1, 0:3]
    S2 = st_ref[1:4, 0:3]
    t = _dot(s1, W)
    u = jnp.sum(W * _dot(S2, W), axis=0, keepdims=True)
    b1 = bp1_ref[...]
    mean = t / CNT + b1
    ex2 = u / CNT + 2.0 * b1 * (t / CNT) + b1 * b1
    var = ex2 - mean * mean
    sc = gp_ref[...] * lax.rsqrt(var + EPS)
    sh = bp_ref[...] - mean * sc
    prf = pr.reshape(pr.shape[0] * NSAMPLE, 3)
    pp = _dot(prf, W) + b1
    ppn = jnp.maximum(pp * sc + sh, 0.0)
    p2 = _dot(ppn, wp2_ref[...]) + bp2_ref[...]
    return p2.reshape(pr.shape[0], NSAMPLE, OUT_PLANES)


def _w_pre(kv_ref, xq_ref, pr_ref, st_ref, wp1_ref, bp1_ref, gp_ref, bp_ref,
           wp2_ref, bp2_ref):
    p = _linear_p(pr_ref[...], st_ref, wp1_ref, bp1_ref, gp_ref, bp_ref,
                  wp2_ref, bp2_ref)
    x_k = kv_ref[:, :, 0:MID]
    w = x_k - xq_ref[...][:, None, :] + p[:, :, 0:MID] + p[:, :, MID:]
    return w, p


_SMALL_SPECS = [
    pl.BlockSpec((8, 8), lambda i: (0, 0)),
    pl.BlockSpec((3, 3), lambda i: (0, 0)),
    pl.BlockSpec((1, 3), lambda i: (0, 0)),
    pl.BlockSpec((1, 3), lambda i: (0, 0)),
    pl.BlockSpec((1, 3), lambda i: (0, 0)),
    pl.BlockSpec((3, OUT_PLANES), lambda i: (0, 0)),
    pl.BlockSpec((1, OUT_PLANES), lambda i: (0, 0)),
]
_NQB = NTOT // QB


def _bn2_kernel(kv_ref, xq_ref, pr_ref, st_ref, wp1_ref, bp1_ref, gp_ref,
                bp_ref, wp2_ref, bp2_ref, wpre_ref, out_ref, s_ref, ss_ref):
    i = pl.program_id(0)

    @pl.when(i == 0)
    def _():
        s_ref[...] = jnp.zeros_like(s_ref)
        ss_ref[...] = jnp.zeros_like(ss_ref)

    w, _ = _w_pre(kv_ref, xq_ref, pr_ref, st_ref, wp1_ref, bp1_ref, gp_ref,
                  bp_ref, wp2_ref, bp2_ref)
    wpre_ref[...] = w
    s_ref[...] += jnp.sum(w, axis=(0, 1))[None, :]
    ss_ref[...] += jnp.sum(w * w, axis=(0, 1))[None, :]

    @pl.when(i == _NQB - 1)
    def _():
        out_ref[...] = jnp.concatenate(
            [s_ref[...], ss_ref[...], jnp.zeros((6, MID), jnp.float32)],
            axis=0)


def _run_bn2(kv, xq, pr, stats, Wp1, bp1, gp, bp, Wp2, bp2):
    return pl.pallas_call(
        _bn2_kernel,
        grid=(_NQB,),
        in_specs=[
            pl.BlockSpec((QB, NSAMPLE, 128), lambda i: (i, 0, 0)),
            pl.BlockSpec((QB, MID), lambda i: (i, 0)),
            pl.BlockSpec((QB, NSAMPLE, 3), lambda i: (i, 0, 0)),
        ] + _SMALL_SPECS,
        out_specs=[
            pl.BlockSpec((QB, NSAMPLE, MID), lambda i: (i, 0, 0)),
            pl.BlockSpec((8, MID), lambda i: (0, 0)),
        ],
        out_shape=[
            jax.ShapeDtypeStruct((NTOT, NSAMPLE, MID), jnp.float32),
            jax.ShapeDtypeStruct((8, MID), jnp.float32),
        ],
        scratch_shapes=[pltpu.VMEM((1, MID), jnp.float32),
                        pltpu.VMEM((1, MID), jnp.float32)],
        compiler_params=pltpu.CompilerParams(
            dimension_semantics=("arbitrary",)),
    )(kv, xq, pr, stats, Wp1, bp1[None, :], gp[None, :], bp[None, :], Wp2,
      bp2[None, :])


def _bn_vec(stats_ref, g_ref, b_ref, width):
    mean = stats_ref[0:1, :] / CNT
    var = stats_ref[1:2, :] / CNT - mean * mean
    sc = g_ref[...] / jnp.sqrt(var + EPS)
    sh = b_ref[...] - mean * sc
    return sc.reshape(1, 1, width), sh.reshape(1, 1, width)


def _w1_kernel(wpre_ref, bn2_ref, gw1_ref, bw1g_ref, ww1_ref,
               bw1_ref, w1_ref, out_ref, s_ref, ss_ref):
    i = pl.program_id(0)

    @pl.when(i == 0)
    def _():
        s_ref[...] = jnp.zeros_like(s_ref)
        ss_ref[...] = jnp.zeros_like(ss_ref)

    sc, sh = _bn_vec(bn2_ref, gw1_ref, bw1g_ref, MID)
    w = jnp.maximum(wpre_ref[...] * sc + sh, 0.0)
    w1 = _dot(w.reshape(QB * NSAMPLE, MID), ww1_ref[...]) + bw1_ref[...]
    w1 = w1.reshape(QB, NSAMPLE, MID // SHARE)
    w1_ref[...] = w1
    s_ref[...] += jnp.sum(w1, axis=(0, 1))[None, :]
    ss_ref[...] += jnp.sum(w1 * w1, axis=(0, 1))[None, :]

    @pl.when(i == _NQB - 1)
    def _():
        out_ref[...] = jnp.concatenate(
            [s_ref[...], ss_ref[...],
             jnp.zeros((6, MID // SHARE), jnp.float32)], axis=0)


def _run_w1(wpre, bn2, gw1, bw1g, Ww1, bw1):
    return pl.pallas_call(
        _w1_kernel,
        grid=(_NQB,),
        in_specs=[
            pl.BlockSpec((QB, NSAMPLE, MID), lambda i: (i, 0, 0)),
            pl.BlockSpec((8, MID), lambda i: (0, 0)),
            pl.BlockSpec((1, MID), lambda i: (0, 0)),
            pl.BlockSpec((1, MID), lambda i: (0, 0)),
            pl.BlockSpec((MID, MID // SHARE), lambda i: (0, 0)),
            pl.BlockSpec((1, MID // SHARE), lambda i: (0, 0)),
        ],
        out_specs=[
            pl.BlockSpec((QB, NSAMPLE, MID // SHARE), lambda i: (i, 0, 0)),
            pl.BlockSpec((8, MID // SHARE), lambda i: (0, 0)),
        ],
        out_shape=[
            jax.ShapeDtypeStruct((NTOT, NSAMPLE, MID // SHARE), jnp.float32),
            jax.ShapeDtypeStruct((8, MID // SHARE), jnp.float32),
        ],
        scratch_shapes=[pltpu.VMEM((1, MID // SHARE), jnp.float32),
                        pltpu.VMEM((1, MID // SHARE), jnp.float32)],
        compiler_params=pltpu.CompilerParams(
            dimension_semantics=("arbitrary",)),
    )(wpre, bn2, gw1[None, :], bw1g[None, :], Ww1, bw1[None, :])


def _out_kernel(kv_ref, pr_ref, w1_ref, st_ref, wp1_ref, bp1_ref, gp_ref,
                bp_ref, wp2_ref, bp2_ref, bn3_ref, gw2_ref, bw2g_ref, ww2_ref,
                bw2_ref, x_ref):
    p = _linear_p(pr_ref[...], st_ref, wp1_ref, bp1_ref, gp_ref, bp_ref,
                  wp2_ref, bp2_ref)
    sc, sh = _bn_vec(bn3_ref, gw2_ref, bw2g_ref, MID // SHARE)
    w1 = jnp.maximum(w1_ref[...] * sc + sh, 0.0)
    w2 = _dot(w1.reshape(QB * NSAMPLE, MID // SHARE), ww2_ref[...])
    w2 = w2.reshape(QB, NSAMPLE, OUT_PLANES // SHARE) + \
        bw2_ref[...].reshape(1, 1, OUT_PLANES // SHARE)
    m = jnp.max(w2, axis=1, keepdims=True)
    e = jnp.exp(w2 - m)
    w = e * (1.0 / jnp.sum(e, axis=1, keepdims=True))
    wtile = jnp.concatenate([w] * SHARE, axis=2)
    x_v = kv_ref[...]
    x_ref[...] = (x_v + p) * wtile


def _run_out(kv, pr, w1, stats, Wp1, bp1, gp, bp, Wp2, bp2, bn3, gw2, bw2g,
             Ww2, bw2):
    return pl.pallas_call(
        _out_kernel,
        grid=(_NQB,),
        in_specs=[
            pl.BlockSpec((QB, NSAMPLE, 128), lambda i: (i, 0, 0)),
            pl.BlockSpec((QB, NSAMPLE, 3), lambda i: (i, 0, 0)),
            pl.BlockSpec((QB, NSAMPLE, MID // SHARE), lambda i: (i, 0, 0)),
        ] + _SMALL_SPECS + [
            pl.BlockSpec((8, MID // SHARE), lambda i: (0, 0)),
            pl.BlockSpec((1, MID // SHARE), lambda i: (0, 0)),
            pl.BlockSpec((1, MID // SHARE), lambda i: (0, 0)),
            pl.BlockSpec((MID // SHARE, OUT_PLANES // SHARE), lambda i: (0, 0)),
            pl.BlockSpec((1, OUT_PLANES // SHARE), lambda i: (0, 0)),
        ],
        out_specs=pl.BlockSpec((QB, NSAMPLE, OUT_PLANES), lambda i: (i, 0, 0)),
        out_shape=jax.ShapeDtypeStruct((NTOT, NSAMPLE, OUT_PLANES),
                                       jnp.float32),
    )(kv, pr, w1, stats, Wp1, bp1[None, :], gp[None, :], bp[None, :], Wp2,
      bp2[None, :], bn3,
      gw2[None, :], bw2g[None, :], Ww2, bw2[None, :])


def kernel(xyz1, xyz2, points1, points2, Wq, bq, Wk, bk, Wv, bv, Wp1, bp1,
           gp, bp, Wp2, bp2, gw1, bw1g, Ww1, bw1, gw2, bw2g, Ww2, bw2):
    t_v, t_kx = _build_kv_table(points2, xyz2, Wk, bk, Wv, bv)
    idx, xq = _run_knn(xyz1, xyz2, points1, Wq, bq)
    idx_flat = idx.reshape(NTOT * NSAMPLE)

    g_kx = _gather_rows(t_kx, idx_flat).reshape(NTOT, NSAMPLE, 128)
    g_v = _gather_rows(t_v, idx_flat).reshape(NTOT, NSAMPLE, 128)

    pr_f, stats = _run_prstats(g_kx, xyz1.reshape(NTOT, 3))
    xq_f = xq.reshape(NTOT, MID)

    wpre, bn2 = _run_bn2(g_kx, xq_f, pr_f, stats, Wp1, bp1, gp, bp, Wp2, bp2)
    w1, bn3 = _run_w1(wpre, bn2, gw1, bw1g, Ww1, bw1)
    x = _run_out(g_v, pr_f, w1, stats, Wp1, bp1, gp, bp, Wp2, bp2, bn3,
                 gw2, bw2g, Ww2, bw2)
    return (x, pr_f)
